# baseline (speedup 1.0000x reference)
import functools
import jax
import jax.numpy as jnp
from jax import lax
from jax.experimental import pallas as pl
from jax.experimental.pallas import tpu as pltpu
from jax.experimental.pallas import tpu_sc as plsc

_V = 1000000
_CH = 2048
_NW = 32
_NCHUNK = _V // _CH
_REM_OFF = _NCHUNK * _CH
_REM = 512
_TAIL_OFF = _REM_OFF + _REM
_TAIL = _V - _TAIL_OFF
_PER_ROW = 2 * _NCHUNK


def _scale_rows(tbuf, wbuf, ncols):
    nv = ncols // 16

    def body(i, _):
        for r in range(8):
            sl = pl.ds(i * 16, 16)
            wv = jnp.maximum(wbuf[r, sl], 0.0) + 1e-9
            tbuf[r, sl] = tbuf[r, sl] * wv
        return 0

    lax.fori_loop(0, nv, body, 0)


def _sc_body(tt, wt, out, tbuf, wbuf):
    cid = lax.axis_index("c")
    sid = lax.axis_index("s")
    wid = sid * 2 + cid

    n0 = (_PER_ROW - wid + _NW - 1) // _NW

    def c0_body(j, _):
        k = j * _NW + wid
        tr = k // _NCHUNK
        off = (k % _NCHUNK) * _CH
        pltpu.sync_copy(tt.at[0, pl.ds(tr * 8, 8), pl.ds(off, _CH)], tbuf)
        pltpu.sync_copy(tbuf, out.at[0, pl.ds(tr * 8, 8), pl.ds(off, _CH)])
        return 0

    lax.fori_loop(0, n0, c0_body, 0)

    def c1_body(j, _):
        k = j * _NW + wid
        tr = k // _NCHUNK
        off = (k % _NCHUNK) * _CH
        rs = pl.ds(tr * 8, 8)
        ls = pl.ds(off, _CH)
        pltpu.sync_copy(tt.at[1, rs, ls], tbuf)
        pltpu.sync_copy(wt.at[rs, ls], wbuf)
        _scale_rows(tbuf, wbuf, _CH)
        pltpu.sync_copy(tbuf, out.at[1, rs, ls])
        return 0

    lax.fori_loop(0, n0, c1_body, 0)

    @pl.when(wid < 4)
    def _rem():
        c = wid // 2
        tr = wid % 2
        rs = pl.ds(tr * 8, 8)
        ls = pl.ds(_REM_OFF, _REM)
        tdst = tbuf.at[:, pl.ds(0, _REM)]
        pltpu.sync_copy(tt.at[c, rs, ls], tdst)

        @pl.when(c == 1)
        def _mul():
            pltpu.sync_copy(wt.at[rs, ls], wbuf.at[:, pl.ds(0, _REM)])
            _scale_rows(tbuf, wbuf, _REM)

        pltpu.sync_copy(tdst, out.at[c, rs, ls])


def _tc_tail_kernel(x_ref, w_ref, t_ref, o_ref):
    del x_ref
    pw = jnp.maximum(w_ref[...], 0.0) + 1e-9
    o_ref[0] = t_ref[0]
    o_ref[1] = t_ref[1] * pw


def kernel(t, weights):
    v, _, width = t.shape
    tt = jnp.transpose(t, (1, 2, 0))
    wt = jnp.transpose(weights, (1, 0))
    sck = pl.kernel(
        _sc_body,
        out_type=jax.ShapeDtypeStruct((2, width, v), jnp.float32),
        mesh=plsc.VectorSubcoreMesh(core_axis_name="c", subcore_axis_name="s"),
        scratch_types=[
            pltpu.VMEM((8, _CH), jnp.float32),
            pltpu.VMEM((8, _CH), jnp.float32),
        ],
        compiler_params=pltpu.CompilerParams(use_tc_tiling_on_sc=True),
    )
    out = sck(tt, wt)

    tb = 128
    ti = _TAIL_OFF // tb
    out = pl.pallas_call(
        _tc_tail_kernel,
        grid=(1,),
        in_specs=[
            pl.BlockSpec((2, width, tb), lambda i: (0, 0, ti)),
            pl.BlockSpec((width, tb), lambda i: (0, ti)),
            pl.BlockSpec((2, width, tb), lambda i: (0, 0, ti)),
        ],
        out_specs=pl.BlockSpec((2, width, tb), lambda i: (0, 0, ti)),
        out_shape=jax.ShapeDtypeStruct((2, width, v), jnp.float32),
        input_output_aliases={0: 0},
    )(out, wt, tt)
    return jnp.transpose(out, (2, 0, 1))
